# baseline (speedup 1.0000x reference)
import jax
import jax.numpy as jnp
from jax.experimental import pallas as pl
from jax.experimental.pallas import tpu as pltpu


def _pos_body(col_ref, row_ref, out_ref, acc_ref):
    @pl.when(pl.program_id(0) == 0)
    def _():
        col_t = col_ref[...].T
        row_t = row_ref[...].T
        for g in range(4):
            acc_ref[:128, pl.ds(32 * g, 32)] = col_t
        lane128 = acc_ref[:128, 0:128]
        for ls in range(1, 8):
            acc_ref[:128, pl.ds(128 * ls, 128)] = lane128
        for i in range(32):
            acc_ref[128:, pl.ds(32 * i, 32)] = jnp.broadcast_to(
                row_t[:, i : i + 1], (128, 32)
            )

    out_ref[0] = acc_ref[...]


@jax.jit
def _pos_embed(row_embed, col_embed):
    out = pl.pallas_call(
        _pos_body,
        grid=(4,),
        in_specs=[
            pl.BlockSpec((32, 128), lambda i: (0, 0)),
            pl.BlockSpec((32, 128), lambda i: (0, 0)),
        ],
        out_specs=pl.BlockSpec((1, 256, 1024), lambda i: (i, 0, 0)),
        out_shape=jax.ShapeDtypeStruct((4, 256, 1024), jnp.float32),
        scratch_shapes=[pltpu.VMEM((256, 1024), jnp.float32)],
    )(col_embed[:32], row_embed[:32])
    return out.reshape(4, 256, 32, 32)


def kernel(x, row_embed, col_embed):
    del x
    return _pos_embed(row_embed, col_embed)


# speedup vs baseline: 1.2739x; 1.2739x over previous
import jax
import jax.numpy as jnp
from jax.experimental import pallas as pl
from jax.experimental.pallas import tpu as pltpu


def _pos_body(col_ref, row_ref, out_ref, acc_ref):
    @pl.when(pl.program_id(0) == 0)
    def _():
        col_t = col_ref[0:32, :].T
        row_t = row_ref[0:32, :].T
        for g in range(4):
            acc_ref[:128, pl.ds(32 * g, 32)] = col_t
        lane128 = acc_ref[:128, 0:128]
        for ls in range(1, 8):
            acc_ref[:128, pl.ds(128 * ls, 128)] = lane128
        for i in range(32):
            acc_ref[128:, pl.ds(32 * i, 32)] = jnp.broadcast_to(
                row_t[:, i : i + 1], (128, 32)
            )

    out_ref[0] = acc_ref[...]


@jax.jit
def _pos_embed(row_embed, col_embed):
    out = pl.pallas_call(
        _pos_body,
        grid=(4,),
        in_specs=[
            pl.BlockSpec(memory_space=pltpu.VMEM),
            pl.BlockSpec(memory_space=pltpu.VMEM),
        ],
        out_specs=pl.BlockSpec((1, 256, 1024), lambda i: (i, 0, 0)),
        out_shape=jax.ShapeDtypeStruct((4, 256, 1024), jnp.float32),
        scratch_shapes=[pltpu.VMEM((256, 1024), jnp.float32)],
    )(col_embed, row_embed)
    return out.reshape(4, 256, 32, 32)


def kernel(x, row_embed, col_embed):
    del x
    return _pos_embed(row_embed, col_embed)
